# baseline (speedup 1.0000x reference)
import functools

import numpy as np
import jax
import jax.numpy as jnp
from jax import lax
from jax.experimental import pallas as pl
from jax.experimental.pallas import tpu as pltpu
from jax.experimental.pallas import tpu_sc as plsc

_ROT_A = (13, 15, 26, 6)
_ROT_B = (17, 29, 16, 24)


def _rotl(x, r):
    return lax.shift_left(x, jnp.uint32(r)) | lax.shift_right_logical(
        x, jnp.uint32(32 - r))


def _sample_body(keys_ref, pw_ref, idx_ref, w_ref, scr_ref):
    bb, n = pw_ref.shape
    nk = idx_ref.shape[1]
    pw = pw_ref[...]
    m = jnp.max(pw, axis=1, keepdims=True)
    lse = jnp.log(jnp.sum(jnp.exp(pw - m), axis=1, keepdims=True)) + m
    winv = jnp.exp(lse - pw)

    w_ref[...] = jnp.full((bb, n), np.float32(np.log(1.0 / n)), jnp.float32)

    rowu = lax.broadcasted_iota(jnp.uint32, (bb, n), 0)
    colu = lax.broadcasted_iota(jnp.uint32, (bb, n), 1)
    base = lax.convert_element_type(pl.program_id(0) * bb, jnp.uint32)
    cnt = (base + rowu) * jnp.uint32(n) + colu
    lane = lax.broadcasted_iota(jnp.int32, (bb, n), 1)
    tiny = np.float32(np.finfo(np.float32).tiny)

    def body(i, carry):
        ks = tuple(
            lax.convert_element_type(keys_ref[i, t], jnp.uint32)
            for t in range(3))
        x0 = ks[0]
        x1 = cnt + ks[1]
        for blk in range(5):
            for r in (_ROT_A if blk % 2 == 0 else _ROT_B):
                x0 = x0 + x1
                x1 = _rotl(x1, r) ^ x0
            x0 = x0 + ks[(blk + 1) % 3]
            x1 = x1 + (ks[(blk + 2) % 3] + jnp.uint32(blk + 1))
        bits = x0 ^ x1
        fb = lax.shift_right_logical(bits, jnp.uint32(9)) | jnp.uint32(
            0x3F800000)
        u = lax.bitcast_convert_type(fb, jnp.float32) - np.float32(1.0)
        t = -jnp.log(u)
        v = t * winv
        am = jnp.argmin(v, axis=1).astype(jnp.int32)
        scr_ref[pl.ds(i, 1), :] = jnp.reshape(am, (1, bb))
        return carry

    lax.fori_loop(0, nk, body, 0, unroll=32)
    idx_ref[...] = jnp.transpose(scr_ref[...], (1, 0))


def _sample_indices(pw, keys3):
    b, n = pw.shape
    bb = 128
    grid = (b // bb,)
    return pl.pallas_call(
        _sample_body,
        grid=grid,
        in_specs=[
            pl.BlockSpec(memory_space=pltpu.SMEM),
            pl.BlockSpec((bb, n), lambda i: (i, 0)),
        ],
        out_specs=[
            pl.BlockSpec((bb, n), lambda i: (i, 0)),
            pl.BlockSpec((bb, n), lambda i: (i, 0)),
        ],
        out_shape=[
            jax.ShapeDtypeStruct((b, n), jnp.int32),
            jax.ShapeDtypeStruct((b, n), jnp.float32),
        ],
        scratch_shapes=[pltpu.VMEM((n, bb), jnp.int32)],
        compiler_params=pltpu.CompilerParams(
            dimension_semantics=("parallel",)),
    )(keys3, pw)


_SC_CORES = 2
_SC_SUBCORES = 16
_SC_WORKERS = _SC_CORES * _SC_SUBCORES


def _gather(states2d, idx2d, b, n, d):
    rows_per_w = b // _SC_WORKERS
    gr = 4
    n_groups = rows_per_w // gr
    n_chunks = n // 16
    mesh = plsc.VectorSubcoreMesh(core_axis_name="c", subcore_axis_name="s")

    @functools.partial(
        pl.kernel,
        mesh=mesh,
        out_type=jax.ShapeDtypeStruct((b, n * d), jnp.float32),
        scratch_types=[
            pltpu.VMEM((gr, n * d), jnp.float32),
            pltpu.VMEM((gr, n), jnp.int32),
            pltpu.VMEM((gr, n * d), jnp.float32),
        ],
        compiler_params=pltpu.CompilerParams(needs_layout_passes=False),
    )
    def gk(st_hbm, idx_hbm, out_hbm, st_v, idx_v, out_v):
        wid = lax.axis_index("s") * _SC_CORES + lax.axis_index("c")
        row0 = wid * rows_per_w
        iota_d = lax.iota(jnp.int32, 16) * d

        def group_body(g, carry):
            row = row0 + g * gr
            pltpu.sync_copy(st_hbm.at[pl.ds(row, gr), :], st_v)
            pltpu.sync_copy(idx_hbm.at[pl.ds(row, gr), :], idx_v)
            for r in range(gr):
                rvec = jnp.full((16,), r, jnp.int32)

                def chunk_body(ch, carry2):
                    i16 = idx_v[r, pl.ds(ch * 16, 16)]
                    pos = i16 * d
                    out_pos = iota_d + ch * (16 * d)
                    for c in range(d):
                        vals = plsc.load_gather(st_v, [rvec, pos + c])
                        plsc.store_scatter(out_v, [rvec, out_pos + c], vals)
                    return carry2

                lax.fori_loop(0, n_chunks, chunk_body, 0)
            pltpu.sync_copy(out_v, out_hbm.at[pl.ds(row, gr), :])
            return carry

        lax.fori_loop(0, n_groups, group_body, 0)

    return gk(states2d, idx2d)


def kernel(particle_states, particle_weights, alpha):
    b, n = particle_weights.shape
    del alpha

    keys = jax.random.key_data(jax.random.split(jax.random.key(42), n))
    k0 = keys[:, 0]
    k1 = keys[:, 1]
    k2 = k0 ^ k1 ^ np.uint32(0x1BD11BDA)
    keys3 = lax.bitcast_convert_type(
        jnp.stack([k0, k1, k2], axis=1), jnp.int32)

    idx_bn, new_weights = _sample_indices(particle_weights, keys3)

    out2d = _gather(particle_states.reshape(b, n * 3), idx_bn, b, n, 3)
    new_states = out2d.reshape(b, n, 3)
    return new_states, new_weights


# speedup vs baseline: 1.0238x; 1.0238x over previous
import functools

import numpy as np
import jax
import jax.numpy as jnp
from jax import lax
from jax.experimental import pallas as pl
from jax.experimental.pallas import tpu as pltpu
from jax.experimental.pallas import tpu_sc as plsc

_ROT_A = (13, 15, 26, 6)
_ROT_B = (17, 29, 16, 24)


def _rotl(x, r):
    return lax.shift_left(x, jnp.uint32(r)) | lax.shift_right_logical(
        x, jnp.uint32(32 - r))


def _sample_body(keys_ref, pw_ref, idx_ref, w_ref, scr_ref):
    bb, n = pw_ref.shape
    nk = idx_ref.shape[1]
    pw = pw_ref[...]
    m = jnp.max(pw, axis=1, keepdims=True)
    lse = jnp.log(jnp.sum(jnp.exp(pw - m), axis=1, keepdims=True)) + m
    winv = jnp.exp(lse - pw)

    w_ref[...] = jnp.full((bb, n), np.float32(np.log(1.0 / n)), jnp.float32)

    rowu = lax.broadcasted_iota(jnp.uint32, (bb, n), 0)
    colu = lax.broadcasted_iota(jnp.uint32, (bb, n), 1)
    base = lax.convert_element_type(pl.program_id(0) * bb, jnp.uint32)
    cnt = (base + rowu) * jnp.uint32(n) + colu
    lane = lax.broadcasted_iota(jnp.int32, (bb, n), 1)
    tiny = np.float32(np.finfo(np.float32).tiny)

    def body(i, carry):
        ks = tuple(
            lax.convert_element_type(keys_ref[i, t], jnp.uint32)
            for t in range(3))
        x0 = ks[0]
        x1 = cnt + ks[1]
        for blk in range(5):
            for r in (_ROT_A if blk % 2 == 0 else _ROT_B):
                x0 = x0 + x1
                x1 = _rotl(x1, r) ^ x0
            x0 = x0 + ks[(blk + 1) % 3]
            x1 = x1 + (ks[(blk + 2) % 3] + jnp.uint32(blk + 1))
        bits = x0 ^ x1
        fb = lax.shift_right_logical(bits, jnp.uint32(9)) | jnp.uint32(
            0x3F800000)
        u = lax.bitcast_convert_type(fb, jnp.float32) - np.float32(1.0)
        t = -jnp.log(u)
        v = t * winv
        am = jnp.argmin(v, axis=1).astype(jnp.int32)
        scr_ref[pl.ds(i, 1), :] = jnp.reshape(am, (1, bb))
        return carry

    lax.fori_loop(0, nk, body, 0, unroll=16)
    idx_ref[...] = jnp.transpose(scr_ref[...], (1, 0))


def _sample_indices(pw, keys3):
    b, n = pw.shape
    bb = 128
    grid = (b // bb,)
    return pl.pallas_call(
        _sample_body,
        grid=grid,
        in_specs=[
            pl.BlockSpec(memory_space=pltpu.SMEM),
            pl.BlockSpec((bb, n), lambda i: (i, 0)),
        ],
        out_specs=[
            pl.BlockSpec((bb, n), lambda i: (i, 0)),
            pl.BlockSpec((bb, n), lambda i: (i, 0)),
        ],
        out_shape=[
            jax.ShapeDtypeStruct((b, n), jnp.int32),
            jax.ShapeDtypeStruct((b, n), jnp.float32),
        ],
        scratch_shapes=[pltpu.VMEM((n, bb), jnp.int32)],
        compiler_params=pltpu.CompilerParams(
            dimension_semantics=("parallel",)),
    )(keys3, pw)


_SC_CORES = 2
_SC_SUBCORES = 16
_SC_WORKERS = _SC_CORES * _SC_SUBCORES


def _gather(states2d, idx2d, b, n, d):
    rows_per_w = b // _SC_WORKERS
    gr = 8
    n_groups = rows_per_w // gr
    n_chunks = n // 16
    mesh = plsc.VectorSubcoreMesh(core_axis_name="c", subcore_axis_name="s")

    @functools.partial(
        pl.kernel,
        mesh=mesh,
        out_type=jax.ShapeDtypeStruct((b, n * d), jnp.float32),
        scratch_types=[
            pltpu.VMEM((gr, n * d), jnp.float32),
            pltpu.VMEM((gr, n), jnp.int32),
            pltpu.VMEM((gr, n * d), jnp.float32),
        ],
        compiler_params=pltpu.CompilerParams(needs_layout_passes=False),
    )
    def gk(st_hbm, idx_hbm, out_hbm, st_v, idx_v, out_v):
        wid = lax.axis_index("s") * _SC_CORES + lax.axis_index("c")
        row0 = wid * rows_per_w
        iota_d = lax.iota(jnp.int32, 16) * d

        def group_body(g, carry):
            row = row0 + g * gr
            pltpu.sync_copy(st_hbm.at[pl.ds(row, gr), :], st_v)
            pltpu.sync_copy(idx_hbm.at[pl.ds(row, gr), :], idx_v)
            for r in range(gr):
                rvec = jnp.full((16,), r, jnp.int32)

                def chunk_body(ch, carry2):
                    i16 = idx_v[r, pl.ds(ch * 16, 16)]
                    pos = i16 * d
                    out_pos = iota_d + ch * (16 * d)
                    for c in range(d):
                        vals = plsc.load_gather(st_v, [rvec, pos + c])
                        plsc.store_scatter(out_v, [rvec, out_pos + c], vals)
                    return carry2

                lax.fori_loop(0, n_chunks, chunk_body, 0)
            pltpu.sync_copy(out_v, out_hbm.at[pl.ds(row, gr), :])
            return carry

        lax.fori_loop(0, n_groups, group_body, 0)

    return gk(states2d, idx2d)


def kernel(particle_states, particle_weights, alpha):
    b, n = particle_weights.shape
    del alpha

    keys = jax.random.key_data(jax.random.split(jax.random.key(42), n))
    k0 = keys[:, 0]
    k1 = keys[:, 1]
    k2 = k0 ^ k1 ^ np.uint32(0x1BD11BDA)
    keys3 = lax.bitcast_convert_type(
        jnp.stack([k0, k1, k2], axis=1), jnp.int32)

    idx_bn, new_weights = _sample_indices(particle_weights, keys3)

    out2d = _gather(particle_states.reshape(b, n * 3), idx_bn, b, n, 3)
    new_states = out2d.reshape(b, n, 3)
    return new_states, new_weights


# speedup vs baseline: 1.0238x; 1.0000x over previous
import functools

import numpy as np
import jax
import jax.numpy as jnp
from jax import lax
from jax.experimental import pallas as pl
from jax.experimental.pallas import tpu as pltpu
from jax.experimental.pallas import tpu_sc as plsc

_ROT_A = (13, 15, 26, 6)
_ROT_B = (17, 29, 16, 24)


def _rotl(x, r):
    return lax.shift_left(x, jnp.uint32(r)) | lax.shift_right_logical(
        x, jnp.uint32(32 - r))


def _sample_body(keys_ref, pw_ref, idx_ref, w_ref, scr_ref):
    bb, n = pw_ref.shape
    nk = idx_ref.shape[1]
    pw = pw_ref[...]
    m = jnp.max(pw, axis=1, keepdims=True)
    lse = jnp.log(jnp.sum(jnp.exp(pw - m), axis=1, keepdims=True)) + m
    winv = jnp.exp(lse - pw)

    w_ref[...] = jnp.full((bb, n), np.float32(np.log(1.0 / n)), jnp.float32)

    rowu = lax.broadcasted_iota(jnp.uint32, (bb, n), 0)
    colu = lax.broadcasted_iota(jnp.uint32, (bb, n), 1)
    base = lax.convert_element_type(pl.program_id(0) * bb, jnp.uint32)
    cnt = (base + rowu) * jnp.uint32(n) + colu

    def body(i, carry):
        ks = tuple(
            lax.convert_element_type(keys_ref[i, t], jnp.uint32)
            for t in range(3))
        x0 = ks[0]
        x1 = cnt + ks[1]
        for blk in range(5):
            for r in (_ROT_A if blk % 2 == 0 else _ROT_B):
                x0 = x0 + x1
                x1 = _rotl(x1, r) ^ x0
            x0 = x0 + ks[(blk + 1) % 3]
            x1 = x1 + (ks[(blk + 2) % 3] + jnp.uint32(blk + 1))
        bits = x0 ^ x1
        fb = lax.shift_right_logical(bits, jnp.uint32(9)) | jnp.uint32(
            0x3F800000)
        u = lax.bitcast_convert_type(fb, jnp.float32) - np.float32(1.0)
        t = -jnp.log(u)
        v = t * winv
        am = jnp.argmin(v, axis=1).astype(jnp.int32)
        scr_ref[pl.ds(i, 1), :] = jnp.reshape(am, (1, bb))
        return carry

    lax.fori_loop(0, nk, body, 0, unroll=16)
    idx_ref[...] = jnp.transpose(scr_ref[...], (1, 0))


def _sample_indices(pw, keys3):
    b, n = pw.shape
    bb = 128
    grid = (b // bb,)
    return pl.pallas_call(
        _sample_body,
        grid=grid,
        in_specs=[
            pl.BlockSpec(memory_space=pltpu.SMEM),
            pl.BlockSpec((bb, n), lambda i: (i, 0)),
        ],
        out_specs=[
            pl.BlockSpec((bb, n), lambda i: (i, 0)),
            pl.BlockSpec((bb, n), lambda i: (i, 0)),
        ],
        out_shape=[
            jax.ShapeDtypeStruct((b, n), jnp.int32),
            jax.ShapeDtypeStruct((b, n), jnp.float32),
        ],
        scratch_shapes=[pltpu.VMEM((n, bb), jnp.int32)],
        compiler_params=pltpu.CompilerParams(
            dimension_semantics=("parallel",)),
    )(keys3, pw)


_SC_CORES = 2
_SC_SUBCORES = 16
_SC_WORKERS = _SC_CORES * _SC_SUBCORES


def _gather(states2d, idx2d, b, n, d):
    rows_per_w = b // _SC_WORKERS
    gr = 8
    n_groups = rows_per_w // gr
    n_chunks = n // 16
    mesh = plsc.VectorSubcoreMesh(core_axis_name="c", subcore_axis_name="s")

    @functools.partial(
        pl.kernel,
        mesh=mesh,
        out_type=jax.ShapeDtypeStruct((b, n * d), jnp.float32),
        scratch_types=[
            pltpu.VMEM((gr, n * d), jnp.float32),
            pltpu.VMEM((gr, n), jnp.int32),
            pltpu.VMEM((gr, n * d), jnp.float32),
        ],
        compiler_params=pltpu.CompilerParams(needs_layout_passes=False),
    )
    def gk(st_hbm, idx_hbm, out_hbm, st_v, idx_v, out_v):
        wid = lax.axis_index("s") * _SC_CORES + lax.axis_index("c")
        row0 = wid * rows_per_w
        iota_d = lax.iota(jnp.int32, 16) * d

        def group_body(g, carry):
            row = row0 + g * gr
            pltpu.sync_copy(st_hbm.at[pl.ds(row, gr), :], st_v)
            pltpu.sync_copy(idx_hbm.at[pl.ds(row, gr), :], idx_v)
            for r in range(gr):
                rvec = jnp.full((16,), r, jnp.int32)

                def chunk_body(ch, carry2):
                    i16 = idx_v[r, pl.ds(ch * 16, 16)]
                    pos = i16 * d
                    out_pos = iota_d + ch * (16 * d)
                    for c in range(d):
                        vals = plsc.load_gather(st_v, [rvec, pos + c])
                        plsc.store_scatter(out_v, [rvec, out_pos + c], vals)
                    return carry2

                lax.fori_loop(0, n_chunks, chunk_body, 0)
            pltpu.sync_copy(out_v, out_hbm.at[pl.ds(row, gr), :])
            return carry

        lax.fori_loop(0, n_groups, group_body, 0)

    return gk(states2d, idx2d)


def kernel(particle_states, particle_weights, alpha):
    b, n = particle_weights.shape
    del alpha

    keys = jax.random.key_data(jax.random.split(jax.random.key(42), n))
    k0 = keys[:, 0]
    k1 = keys[:, 1]
    k2 = k0 ^ k1 ^ np.uint32(0x1BD11BDA)
    keys3 = lax.bitcast_convert_type(
        jnp.stack([k0, k1, k2], axis=1), jnp.int32)

    idx_bn, new_weights = _sample_indices(particle_weights, keys3)

    out2d = _gather(particle_states.reshape(b, n * 3), idx_bn, b, n, 3)
    new_states = out2d.reshape(b, n, 3)
    return new_states, new_weights
